# initial kernel scaffold (unmeasured)
import jax
import jax.numpy as jnp
from jax import lax
from jax.experimental import pallas as pl
from jax.experimental.pallas import tpu as pltpu

B, H, D, BS = 16, 16, 64, 16
NB = 128
P_LOC = 128
KLOC = P_LOC * BS
SCALE = D ** -0.5
NEG = -1e30


def kernel(Q, K, V, bt, lens):
    lens2 = lens.reshape(B, 1).astype(jnp.int32)

    def body(q_ref, k_ref, v_ref, bt_ref, lens_ref, out_ref,
             comm_ref, send_sem, recv_sem):
        my_x = lax.axis_index("x")
        my_y = lax.axis_index("y")

        q = q_ref[:, 0, :, :]
        k = k_ref[...].reshape(KLOC, H, D)
        v = v_ref[...].reshape(KLOC, H, D)

        valid = lax.broadcasted_iota(jnp.int32, (B, NB), 1) < lens_ref[...]
        pg = lax.broadcasted_iota(jnp.int32, (B, NB, P_LOC), 2) + my_y * P_LOC
        hit = (bt_ref[...][:, :, None] == pg) & valid[:, :, None]
        cnt = jnp.sum(hit.astype(jnp.float32), axis=1)
        ck = jnp.broadcast_to(cnt[:, :, None], (B, P_LOC, BS)).reshape(B, KLOC)

        s = jnp.einsum("bhd,khd->bhk", q, k,
                       preferred_element_type=jnp.float32) * SCALE
        s = jnp.where((ck > 0.0)[:, None, :], s, jnp.float32(NEG))
        m_loc = jnp.max(s, axis=-1)
        p = jnp.exp(s - m_loc[:, :, None]) * ck[:, None, :]
        l_loc = jnp.sum(p, axis=-1)
        acc = jnp.einsum("bhk,khd->bhd", p, v,
                         preferred_element_type=jnp.float32)

        comm_ref[0, :, :, 0:D] = acc
        comm_ref[0, :, :, D:D + 1] = m_loc[:, :, None]
        comm_ref[0, :, :, D + 1:D + 2] = l_loc[:, :, None]

        barrier = pltpu.get_barrier_semaphore()
        pl.semaphore_signal(barrier, inc=1, device_id=(my_x, 1 - my_y),
                            device_id_type=pl.DeviceIdType.MESH)
        pl.semaphore_wait(barrier, 1)

        rdma = pltpu.make_async_remote_copy(
            src_ref=comm_ref.at[0],
            dst_ref=comm_ref.at[1],
            send_sem=send_sem,
            recv_sem=recv_sem,
            device_id=(my_x, 1 - my_y),
            device_id_type=pl.DeviceIdType.MESH,
        )
        rdma.start()
        rdma.wait()

        nb = comm_ref[1]
        acc_n = nb[:, :, 0:D]
        m_n = nb[:, :, D]
        l_n = nb[:, :, D + 1]
        m_tot = jnp.maximum(m_loc, m_n)
        w_a = jnp.exp(m_loc - m_tot)
        w_b = jnp.exp(m_n - m_tot)
        l_tot = l_loc * w_a + l_n * w_b
        out = (acc * w_a[:, :, None] + acc_n * w_b[:, :, None]) / l_tot[:, :, None]
        out_ref[:, 0, :, :] = out

    return pl.pallas_call(
        body,
        out_shape=jax.ShapeDtypeStruct((B, 1, H, D), jnp.float32),
        in_specs=[pl.BlockSpec(memory_space=pltpu.VMEM)] * 5,
        out_specs=pl.BlockSpec(memory_space=pltpu.VMEM),
        scratch_shapes=[
            pltpu.VMEM((2, B, H, D + 2), jnp.float32),
            pltpu.SemaphoreType.DMA,
            pltpu.SemaphoreType.DMA,
        ],
        compiler_params=pltpu.CompilerParams(collective_id=0),
    )(Q, K, V, bt, lens2)


# baseline (device time: 70171 ns/iter reference)
import jax
import jax.numpy as jnp
from jax import lax
from jax.experimental import pallas as pl
from jax.experimental.pallas import tpu as pltpu

B, H, D, BS = 16, 16, 64, 16
NB = 128
P_LOC = 128
KLOC = P_LOC * BS
SCALE = D ** -0.5
NEG = -1e30


def kernel(Q, K, V, bt, lens):
    lens2 = lens.reshape(B, 1).astype(jnp.int32)

    def body(q_ref, k_ref, v_ref, bt_ref, lens_ref, out_ref,
             comm_ref, send_sem, recv_sem):
        my_x = lax.axis_index("x")
        my_y = lax.axis_index("y")

        valid = lax.broadcasted_iota(jnp.int32, (B, NB), 1) < lens_ref[...]
        bt_eff = jnp.where(valid, bt_ref[...], -1)
        pg3 = (lax.broadcasted_iota(jnp.int32, (B, P_LOC, NB), 1)
               + my_y * P_LOC)
        hit = pg3 == jnp.reshape(bt_eff, (B, 1, NB))
        cnt = jnp.sum(hit.astype(jnp.float32), axis=2)
        expand = (lax.broadcasted_iota(jnp.int32, (P_LOC, KLOC), 0)
                  == lax.broadcasted_iota(jnp.int32, (P_LOC, KLOC), 1) // BS
                  ).astype(jnp.float32)
        ck = lax.dot_general(cnt, expand, (((1,), (0,)), ((), ())),
                             preferred_element_type=jnp.float32)
        dead = ck == 0.0

        for h in range(H):
            q_h = q_ref[:, 0, h, :]
            k_h = k_ref[:, :, h, :].reshape(KLOC, D)
            v_h = v_ref[:, :, h, :].reshape(KLOC, D)
            s_h = lax.dot_general(q_h, k_h, (((1,), (1,)), ((), ())),
                                  preferred_element_type=jnp.float32) * SCALE
            s_h = jnp.where(dead, jnp.float32(NEG), s_h)
            m_h = jnp.max(s_h, axis=1, keepdims=True)
            p_h = jnp.exp(s_h - m_h) * ck
            l_h = jnp.sum(p_h, axis=1, keepdims=True)
            acc_h = lax.dot_general(p_h, v_h, (((1,), (0,)), ((), ())),
                                    preferred_element_type=jnp.float32)
            comm_ref[0, :, h, 0:D] = acc_h
            comm_ref[0, :, h, D:D + 1] = m_h
            comm_ref[0, :, h, D + 1:D + 2] = l_h

        barrier = pltpu.get_barrier_semaphore()
        pl.semaphore_signal(barrier, inc=1, device_id=(my_x, 1 - my_y),
                            device_id_type=pl.DeviceIdType.MESH)
        pl.semaphore_wait(barrier, 1)

        rdma = pltpu.make_async_remote_copy(
            src_ref=comm_ref.at[0],
            dst_ref=comm_ref.at[1],
            send_sem=send_sem,
            recv_sem=recv_sem,
            device_id=(my_x, 1 - my_y),
            device_id_type=pl.DeviceIdType.MESH,
        )
        rdma.start()
        rdma.wait()

        for h in range(H):
            acc_a = comm_ref[0, :, h, 0:D]
            m_a = comm_ref[0, :, h, D:D + 1]
            l_a = comm_ref[0, :, h, D + 1:D + 2]
            acc_b = comm_ref[1, :, h, 0:D]
            m_b = comm_ref[1, :, h, D:D + 1]
            l_b = comm_ref[1, :, h, D + 1:D + 2]
            m_tot = jnp.maximum(m_a, m_b)
            w_a = jnp.exp(m_a - m_tot)
            w_b = jnp.exp(m_b - m_tot)
            l_tot = l_a * w_a + l_b * w_b
            out_ref[:, 0, h, :] = (acc_a * w_a + acc_b * w_b) / l_tot

    return pl.pallas_call(
        body,
        out_shape=jax.ShapeDtypeStruct((B, 1, H, D), jnp.float32),
        in_specs=[pl.BlockSpec(memory_space=pltpu.VMEM)] * 5,
        out_specs=pl.BlockSpec(memory_space=pltpu.VMEM),
        scratch_shapes=[
            pltpu.VMEM((2, B, H, D + 2), jnp.float32),
            pltpu.SemaphoreType.DMA,
            pltpu.SemaphoreType.DMA,
        ],
        compiler_params=pltpu.CompilerParams(collective_id=0),
    )(Q, K, V, bt, lens2)


# device time: 52503 ns/iter; 1.3365x vs baseline; 1.3365x over previous
import jax
import jax.numpy as jnp
from jax import lax
from jax.experimental import pallas as pl
from jax.experimental.pallas import tpu as pltpu

B, H, D, BS = 16, 16, 64, 16
NB = 128
P_LOC = 128
KLOC = P_LOC * BS
HH = H // 2
SCALE = D ** -0.5
NEG = -1e30


def kernel(Q, K, V, bt, lens):
    lens2 = lens.reshape(B, 1).astype(jnp.int32)

    def body(q_ref, k_any, v_any, bt_ref, lens_ref, out_ref,
             kscr, vscr, comm_ref, fin_ref,
             kd_sems, vd_sems, ysend, yrecv, xsend, xrecv):
        my_x = lax.axis_index("x")
        my_y = lax.axis_index("y")

        def issue(h0):
            for hh in range(HH):
                pltpu.make_async_copy(
                    k_any.at[:, :, h0 + hh, :], kscr.at[hh], kd_sems.at[hh]
                ).start()
                pltpu.make_async_copy(
                    v_any.at[:, :, h0 + hh, :], vscr.at[hh], vd_sems.at[hh]
                ).start()

        @pl.when(my_x == 0)
        def _():
            issue(0)

        @pl.when(my_x == 1)
        def _():
            issue(HH)

        barrier = pltpu.get_barrier_semaphore()
        pl.semaphore_signal(barrier, inc=1, device_id=(my_x, 1 - my_y),
                            device_id_type=pl.DeviceIdType.MESH)
        pl.semaphore_signal(barrier, inc=1, device_id=(1 - my_x, my_y),
                            device_id_type=pl.DeviceIdType.MESH)
        pl.semaphore_wait(barrier, 2)

        with jax.named_scope("counts"):
            valid = lax.broadcasted_iota(jnp.int32, (B, NB), 1) < lens_ref[...]
            bt_eff = jnp.where(valid, bt_ref[...], -1)
            pg3 = (lax.broadcasted_iota(jnp.int32, (B, P_LOC, NB), 1)
                   + my_y * P_LOC)
            hit = pg3 == jnp.reshape(bt_eff, (B, 1, NB))
            cnt = jnp.sum(hit.astype(jnp.float32), axis=2)
            expand = (lax.broadcasted_iota(jnp.int32, (P_LOC, KLOC), 0)
                      == lax.broadcasted_iota(jnp.int32, (P_LOC, KLOC), 1) // BS
                      ).astype(jnp.float32)
            ck = lax.dot_general(cnt, expand, (((1,), (0,)), ((), ())),
                                 preferred_element_type=jnp.float32)
            dead = ck == 0.0

        def attn_half(h0):
            for hh in range(HH):
                with jax.named_scope(f"attn#hh={hh}"):
                    h = h0 + hh
                    pltpu.make_async_copy(
                        k_any.at[:, :, h, :], kscr.at[hh], kd_sems.at[hh]
                    ).wait()
                    q_h = q_ref[:, 0, h, :]
                    k_h = kscr[hh].reshape(KLOC, D)
                    s_h = lax.dot_general(q_h, k_h, (((1,), (1,)), ((), ())),
                                          preferred_element_type=jnp.float32) * SCALE
                    s_h = jnp.where(dead, jnp.float32(NEG), s_h)
                    m_h = jnp.max(s_h, axis=1, keepdims=True)
                    p_h = jnp.exp(s_h - m_h) * ck
                    l_h = jnp.sum(p_h, axis=1, keepdims=True)
                    pltpu.make_async_copy(
                        v_any.at[:, :, h, :], vscr.at[hh], vd_sems.at[hh]
                    ).wait()
                    v_h = vscr[hh].reshape(KLOC, D)
                    acc_h = lax.dot_general(p_h, v_h, (((1,), (0,)), ((), ())),
                                            preferred_element_type=jnp.float32)
                    comm_ref[0, :, hh, 0:D] = acc_h
                    comm_ref[0, :, hh, D:D + 1] = m_h
                    comm_ref[0, :, hh, D + 1:D + 2] = l_h

        @pl.when(my_x == 0)
        def _():
            attn_half(0)

        @pl.when(my_x == 1)
        def _():
            attn_half(HH)

        with jax.named_scope("y_exchange"):
            yrdma = pltpu.make_async_remote_copy(
                src_ref=comm_ref.at[0],
                dst_ref=comm_ref.at[1],
                send_sem=ysend,
                recv_sem=yrecv,
                device_id=(my_x, 1 - my_y),
                device_id_type=pl.DeviceIdType.MESH,
            )
            yrdma.start()
            yrdma.wait()

        with jax.named_scope("combine"):
            for hh in range(HH):
                acc_a = comm_ref[0, :, hh, 0:D]
                m_a = comm_ref[0, :, hh, D:D + 1]
                l_a = comm_ref[0, :, hh, D + 1:D + 2]
                acc_b = comm_ref[1, :, hh, 0:D]
                m_b = comm_ref[1, :, hh, D:D + 1]
                l_b = comm_ref[1, :, hh, D + 1:D + 2]
                m_tot = jnp.maximum(m_a, m_b)
                w_a = jnp.exp(m_a - m_tot)
                w_b = jnp.exp(m_b - m_tot)
                l_tot = l_a * w_a + l_b * w_b
                fin_ref[0, :, hh, :] = (acc_a * w_a + acc_b * w_b) / l_tot

        with jax.named_scope("x_exchange"):
            xrdma = pltpu.make_async_remote_copy(
                src_ref=fin_ref.at[0],
                dst_ref=fin_ref.at[1],
                send_sem=xsend,
                recv_sem=xrecv,
                device_id=(1 - my_x, my_y),
                device_id_type=pl.DeviceIdType.MESH,
            )
            xrdma.start()
            xrdma.wait()

        @pl.when(my_x == 0)
        def _():
            out_ref[:, 0, 0:HH, :] = fin_ref[0]
            out_ref[:, 0, HH:H, :] = fin_ref[1]

        @pl.when(my_x == 1)
        def _():
            out_ref[:, 0, HH:H, :] = fin_ref[0]
            out_ref[:, 0, 0:HH, :] = fin_ref[1]

    return pl.pallas_call(
        body,
        out_shape=jax.ShapeDtypeStruct((B, 1, H, D), jnp.float32),
        in_specs=[
            pl.BlockSpec(memory_space=pltpu.VMEM),
            pl.BlockSpec(memory_space=pltpu.MemorySpace.HBM),
            pl.BlockSpec(memory_space=pltpu.MemorySpace.HBM),
            pl.BlockSpec(memory_space=pltpu.VMEM),
            pl.BlockSpec(memory_space=pltpu.VMEM),
        ],
        out_specs=pl.BlockSpec(memory_space=pltpu.VMEM),
        scratch_shapes=[
            pltpu.VMEM((HH, P_LOC, BS, D), jnp.float32),
            pltpu.VMEM((HH, P_LOC, BS, D), jnp.float32),
            pltpu.VMEM((2, B, HH, D + 2), jnp.float32),
            pltpu.VMEM((2, B, HH, D), jnp.float32),
            pltpu.SemaphoreType.DMA((HH,)),
            pltpu.SemaphoreType.DMA((HH,)),
            pltpu.SemaphoreType.DMA,
            pltpu.SemaphoreType.DMA,
            pltpu.SemaphoreType.DMA,
            pltpu.SemaphoreType.DMA,
        ],
        compiler_params=pltpu.CompilerParams(collective_id=0),
    )(Q, K, V, bt, lens2)
